# baseline (speedup 1.0000x reference)
import jax
import jax.numpy as jnp
from jax.experimental import pallas as pl


def _copy_body(x_ref, o_ref):
    o_ref[...] = x_ref[...]


def kernel(x):
    return pl.pallas_call(
        _copy_body,
        out_shape=jax.ShapeDtypeStruct((5, 2, 2), jnp.float32),
    )(x)


# speedup vs baseline: 1.0377x; 1.0377x over previous
import jax
import jax.numpy as jnp
from jax.experimental import pallas as pl
from jax.experimental.pallas import tpu as pltpu


def _dma_body(x_hbm, o_hbm, sem):
    copy = pltpu.make_async_copy(x_hbm, o_hbm, sem)
    copy.start()
    copy.wait()


def kernel(x):
    return pl.pallas_call(
        _dma_body,
        in_specs=[pl.BlockSpec(memory_space=pl.ANY)],
        out_specs=pl.BlockSpec(memory_space=pl.ANY),
        out_shape=jax.ShapeDtypeStruct((5, 2, 2), jnp.float32),
        scratch_shapes=[pltpu.SemaphoreType.DMA],
    )(x)
